# baseline (speedup 1.0000x reference)
import functools

import jax
import jax.numpy as jnp
from jax import lax
from jax.experimental import pallas as pl
from jax.experimental.pallas import tpu as pltpu
from jax.experimental.pallas import tpu_sc as plsc

N = 10000
E = 160000
DIN = 256
DOUT = 256
EFD = 16
TASK = 2
HALF = 128
NPAD = N + 8
NH = N // 2
NHP = NH + 8
NRP = 10240
NTW = 640

NS = 16
STRIPE = E // NS
CCH = 2000
GCH = 96
CAP = STRIPE + 32
SB = GCH + 160
ZR = 1000
SHIFT = 14
SENT = (1 << SHIFT) - 1


def _mm_node_body(nf_ref, w_ref, a_ref, b_ref, d_ref):
    res = jnp.dot(nf_ref[...], w_ref[...], preferred_element_type=jnp.float32)
    a_ref[0] = res[:, 0:HALF]
    a_ref[1] = res[:, HALF:2 * HALF]
    b_ref[0] = res[:, 2 * HALF:3 * HALF]
    b_ref[1] = res[:, 3 * HALF:4 * HALF]
    d_ref[...] = res[:, 4 * HALF:6 * HALF]


def _mm_edge_body(ef_ref, w_ref, c_ref):
    c_ref[...] = jnp.dot(ef_ref[...], w_ref[...],
                         preferred_element_type=jnp.float32)


def _node_out_body(d_ref, r0_ref, r1_ref, wb_ref, recv_ref, o_ref):
    z = d_ref[...]
    z = z + jnp.dot(r0_ref[0], wb_ref[0:HALF, :],
                    preferred_element_type=jnp.float32)
    z = z + jnp.dot(r1_ref[0], wb_ref[HALF:2 * HALF, :],
                    preferred_element_type=jnp.float32)
    z = jnp.maximum(z, z * jnp.float32(0.01))
    o_ref[...] = jnp.where(recv_ref[...] > 0.0, z, jnp.float32(0.0))


def _recv_red_body(recv_ref, o_ref):
    o_ref[...] = jnp.max(recv_ref[...], axis=0)[:, None]


def _sc_body(src_hbm, dst_hbm, nt_hbm, a2_hbm, b2_hbm, c2_hbm, zrow_hbm,
             red2_hbm, recvt_hbm,
             nt_v, srcch_v, dstch_v, pk_l, eid_l,
             aidx_b, dv_b, eid_b, bidx, sidx2, recv_v,
             abuf, bbuf, cbuf, red_s, sem_a, sem_b, sem_c):
    c = lax.axis_index("c")
    s = lax.axis_index("s")
    lane = lax.iota(jnp.int32, 16)
    cNP = c * NPAD
    cE = c * E
    ones16 = jnp.full((16,), 1.0, jnp.float32)
    zero16f = jnp.zeros((16,), jnp.float32)

    pltpu.sync_copy(nt_hbm, nt_v)

    def z_body(i, _):
        recv_v[pl.ds(i * 16, 16)] = zero16f
        return 0

    lax.fori_loop(0, NRP // 16, z_body, 0)

    stripe_base = s * STRIPE

    def chunk_body(j, kk):
        ebase = stripe_base + j * CCH
        pltpu.sync_copy(src_hbm.at[pl.ds(ebase, CCH)], srcch_v)
        pltpu.sync_copy(dst_hbm.at[pl.ds(ebase, CCH)], dstch_v)

        def vec_body(i, kk):
            srcv = srcch_v[pl.ds(i * 16, 16)]
            dstv = dstch_v[pl.ds(i * 16, 16)]
            w = plsc.load_gather(nt_v, [jnp.right_shift(srcv, 4)])
            sh = jnp.left_shift(jnp.bitwise_and(srcv, 15), 1)
            nt = jnp.bitwise_and(lax.shift_right_logical(w, sh), 3)
            m = nt == TASK
            plsc.store_scatter(recv_v, [dstv], ones16, mask=m)
            eidv = (ebase + i * 16) + lane
            pkv = jnp.bitwise_or(jnp.left_shift(srcv, SHIFT), dstv)
            plsc.store_compressed(pk_l.at[pl.ds(kk, 16)], pkv, mask=m)
            plsc.store_compressed(eid_l.at[pl.ds(kk, 16)], eidv, mask=m)
            return kk + jnp.sum(m.astype(jnp.int32))

        return lax.fori_loop(0, CCH // 16, vec_body, kk)

    kk = lax.fori_loop(0, STRIPE // CCH, chunk_body, jnp.int32(0))

    pk_l[pl.ds(kk, 16)] = jnp.full((16,), SENT, jnp.int32)
    eid_l[pl.ds(kk, 16)] = jnp.zeros((16,), jnp.int32)

    @pl.when(c == 0)
    def _():
        pltpu.sync_copy(recv_v, recvt_hbm.at[pl.ds(s * NRP, NRP)])

    nv = (kk + 15) // 16

    for p in range(2):
        @pl.when(s < NH // ZR)
        def _():
            pltpu.sync_copy(zrow_hbm, red_s.at[pl.ds(s * ZR, ZR)])

        plsc.subcore_barrier()
        dbase = p * NH + cNP

        def batch():
            for t in range(GCH // 16):
                sl = pl.ds(t * 16, 16)
                dv = dv_b[sl]
                sidx2[0, sl] = dv
                bidx[sl] = dv + dbase
            ca = pltpu.async_copy(a2_hbm.at[aidx_b.at[pl.ds(0, GCH)]],
                                  abuf, sem_a)
            cb = pltpu.async_copy(b2_hbm.at[bidx], bbuf, sem_b)
            cc = pltpu.async_copy(c2_hbm.at[eid_b.at[pl.ds(0, GCH)]],
                                  cbuf, sem_c)
            ca.wait()
            cb.wait()
            cc.wait()

            def row_body(i, _):
                for k2 in range(HALF // 16):
                    sl = pl.ds(k2 * 16, 16)
                    v = abuf[i, sl] + bbuf[i, sl] + cbuf[i, sl]
                    abuf[i, sl] = jnp.maximum(v, v * jnp.float32(0.01))
                return 0

            lax.fori_loop(0, GCH, row_body, 0)
            pltpu.sync_copy(abuf, red_s.at[sidx2.at[0]], add=True)

        def cnt_body(i, kp):
            pkv = pk_l[pl.ds(i * 16, 16)]
            dv = jnp.bitwise_and(pkv, SENT)
            dvl = dv - p * NH
            mr = (dvl >= 0) & (dvl < NH)
            return kp + jnp.sum(mr.astype(jnp.int32))

        kp = lax.fori_loop(0, nv, cnt_body, jnp.int32(0))
        nbatches = (kp + GCH - 1) // GCH

        def b_body(_, st):
            i, fill = st

            def f_cond(st2):
                i2, f2 = st2
                return (f2 < GCH) & (i2 < nv)

            def f_step(st2):
                i2, f2 = st2
                pkv = pk_l[pl.ds(i2 * 16, 16)]
                ev = eid_l[pl.ds(i2 * 16, 16)]
                sv = jnp.right_shift(pkv, SHIFT)
                dv = jnp.bitwise_and(pkv, SENT)
                dvl = dv - p * NH
                mr = (dvl >= 0) & (dvl < NH)
                plsc.store_compressed(aidx_b.at[pl.ds(f2, 16)], sv + cNP,
                                      mask=mr)
                plsc.store_compressed(dv_b.at[pl.ds(f2, 16)], dvl, mask=mr)
                plsc.store_compressed(eid_b.at[pl.ds(f2, 16)],
                                      jnp.left_shift(ev, 1) + c, mask=mr)
                return (i2 + 1, f2 + jnp.sum(mr.astype(jnp.int32)))

            i, fill = lax.while_loop(f_cond, f_step, (i, fill))

            for t in range(GCH // 16):
                aidx_b[pl.ds(fill + t * 16, 16)] = jnp.full((16,), cNP,
                                                            jnp.int32)
                dv_b[pl.ds(fill + t * 16, 16)] = jnp.full((16,), NH,
                                                          jnp.int32)
                eid_b[pl.ds(fill + t * 16, 16)] = jnp.full((16,), cE,
                                                           jnp.int32)
            batch()
            aidx_b[pl.ds(0, 16)] = aidx_b[pl.ds(GCH, 16)]
            dv_b[pl.ds(0, 16)] = dv_b[pl.ds(GCH, 16)]
            eid_b[pl.ds(0, 16)] = eid_b[pl.ds(GCH, 16)]
            return (i, jnp.maximum(fill - GCH, 0))

        lax.fori_loop(0, nbatches, b_body, (jnp.int32(0), jnp.int32(0)))

        plsc.subcore_barrier()

        @pl.when(s < NH // ZR)
        def _():
            pltpu.sync_copy(red_s.at[pl.ds(s * ZR, ZR)],
                            red2_hbm.at[pl.ds(c * NPAD + p * NH + s * ZR,
                                              ZR)])

        plsc.subcore_barrier()


def _make_sc_kernel():
    mesh = plsc.VectorSubcoreMesh(core_axis_name="c", subcore_axis_name="s",
                                  num_cores=2, num_subcores=NS)
    return pl.kernel(
        _sc_body,
        out_type=(
            jax.ShapeDtypeStruct((2 * NPAD, HALF), jnp.float32),
            jax.ShapeDtypeStruct((NS * NRP,), jnp.float32),
        ),
        mesh=mesh,
        compiler_params=pltpu.CompilerParams(needs_layout_passes=False),
        scratch_types=[
            pltpu.VMEM((NTW,), jnp.int32),
            pltpu.VMEM((CCH,), jnp.int32),
            pltpu.VMEM((CCH,), jnp.int32),
            pltpu.VMEM((CAP,), jnp.int32),
            pltpu.VMEM((CAP,), jnp.int32),
            pltpu.VMEM((SB,), jnp.int32),
            pltpu.VMEM((SB,), jnp.int32),
            pltpu.VMEM((SB,), jnp.int32),
            pltpu.VMEM((GCH,), jnp.int32),
            pltpu.VMEM((1, GCH), jnp.int32),
            pltpu.VMEM((NRP,), jnp.float32),
            pltpu.VMEM((GCH, HALF), jnp.float32),
            pltpu.VMEM((GCH, HALF), jnp.float32),
            pltpu.VMEM((GCH, HALF), jnp.float32),
            pltpu.VMEM_SHARED((NHP, HALF), jnp.float32),
            pltpu.SemaphoreType.DMA,
            pltpu.SemaphoreType.DMA,
            pltpu.SemaphoreType.DMA,
        ],
    )


_BN = 400
_BE = 2000
_BC = 1000


@jax.jit
def kernel(nf, ef, edge_index, node_type, W_edge, W_node):
    src = edge_index[0]
    dst = edge_index[1]
    w_cat = jnp.concatenate(
        [W_edge[0:DIN], W_edge[DIN:2 * DIN], W_node[0:DOUT]], axis=1)
    w_ef = W_edge[2 * DIN:2 * DIN + EFD]
    w_bot = W_node[DOUT:DOUT + DIN]

    ntb = node_type.astype(jnp.int32).reshape(N // 16, 16)
    ntp = jnp.sum(ntb << (2 * jnp.arange(16, dtype=jnp.int32)), axis=1)
    ntp = jnp.concatenate([ntp, jnp.zeros((NTW - N // 16,), jnp.int32)])

    a2, b2, d = pl.pallas_call(
        _mm_node_body,
        grid=(N // _BN,),
        in_specs=[
            pl.BlockSpec((_BN, DIN), lambda i: (i, 0)),
            pl.BlockSpec((DIN, 2 * DIN + DOUT), lambda i: (0, 0)),
        ],
        out_specs=[
            pl.BlockSpec((2, _BN, HALF), lambda i: (0, i, 0)),
            pl.BlockSpec((2, _BN, HALF), lambda i: (0, i, 0)),
            pl.BlockSpec((_BN, DOUT), lambda i: (i, 0)),
        ],
        out_shape=[
            jax.ShapeDtypeStruct((2, NPAD, HALF), jnp.float32),
            jax.ShapeDtypeStruct((2, NPAD, HALF), jnp.float32),
            jax.ShapeDtypeStruct((N, DOUT), jnp.float32),
        ],
    )(nf, w_cat)

    ef8 = ef.reshape(E // 8, 8 * EFD)
    wbd = jnp.kron(jnp.eye(8, dtype=jnp.float32), w_ef)
    c2 = pl.pallas_call(
        _mm_edge_body,
        grid=(E // 8 // _BC,),
        in_specs=[
            pl.BlockSpec((_BC, 8 * EFD), lambda i: (i, 0)),
            pl.BlockSpec((8 * EFD, 8 * DOUT), lambda i: (0, 0)),
        ],
        out_specs=pl.BlockSpec((_BC, 8 * DOUT), lambda i: (i, 0)),
        out_shape=jax.ShapeDtypeStruct((E // 8, 8 * DOUT), jnp.float32),
    )(ef8, wbd)
    c2 = c2.reshape(2 * E, HALF)

    a2 = a2.reshape(2 * NPAD, HALF)
    b2 = b2.reshape(2 * NPAD, HALF)

    zrow = jnp.zeros((ZR, HALF), jnp.float32)
    red2, recvt = _make_sc_kernel()(
        src, dst, ntp, a2, b2, c2, zrow)

    recv = pl.pallas_call(
        _recv_red_body,
        grid=(1,),
        in_specs=[pl.BlockSpec((NS, NRP), lambda i: (0, 0))],
        out_specs=pl.BlockSpec((NRP, 1), lambda i: (0, 0)),
        out_shape=jax.ShapeDtypeStruct((NRP, 1), jnp.float32),
    )(recvt.reshape(NS, NRP))

    out = pl.pallas_call(
        _node_out_body,
        grid=(N // _BN,),
        in_specs=[
            pl.BlockSpec((_BN, DOUT), lambda i: (i, 0)),
            pl.BlockSpec((1, _BN, HALF), lambda i: (0, i, 0)),
            pl.BlockSpec((1, _BN, HALF), lambda i: (1, i, 0)),
            pl.BlockSpec((DIN, DOUT), lambda i: (0, 0)),
            pl.BlockSpec((_BN, 1), lambda i: (i, 0)),
        ],
        out_specs=pl.BlockSpec((_BN, DOUT), lambda i: (i, 0)),
        out_shape=jax.ShapeDtypeStruct((N, DOUT), jnp.float32),
    )(d, red2.reshape(2, NPAD, HALF), red2.reshape(2, NPAD, HALF), w_bot,
      recv[:N])
    return out


# speedup vs baseline: 1.5797x; 1.5797x over previous
import functools

import jax
import jax.numpy as jnp
from jax import lax
from jax.experimental import pallas as pl
from jax.experimental.pallas import tpu as pltpu
from jax.experimental.pallas import tpu_sc as plsc

N = 10000
E = 160000
DIN = 256
DOUT = 256
EFD = 16
TASK = 2
HALF = 128
NPAD = N + 8
NH = N // 2
NHP = NH + 8
NRP = 10240
NTW = 640

NS = 16
STRIPE = E // NS
CCH = 2000
GCH = 96
CAP = STRIPE + 32
SB = GCH + 160
ZR = 1000
SHIFT = 14
SENT = (1 << SHIFT) - 1


def _mm_node_body(nf_ref, w_ref, a_ref, b_ref, d_ref):
    res = jnp.dot(nf_ref[...], w_ref[...], preferred_element_type=jnp.float32)
    a_ref[0] = res[:, 0:HALF]
    a_ref[1] = res[:, HALF:2 * HALF]
    b_ref[0] = res[:, 2 * HALF:3 * HALF]
    b_ref[1] = res[:, 3 * HALF:4 * HALF]
    d_ref[...] = res[:, 4 * HALF:6 * HALF]


def _mm_edge_body(ef_ref, w_ref, ct_ref, cb_ref):
    res = lax.dot_general(ef_ref[...], w_ref[...],
                          (((0,), (0,)), ((), ())),
                          preferred_element_type=jnp.float32)
    ct_ref[...] = res[:, 0:HALF]
    cb_ref[...] = res[:, HALF:2 * HALF]


def _node_out_body(d_ref, r0_ref, r1_ref, wb_ref, recv_ref, o_ref):
    z = d_ref[...]
    z = z + jnp.dot(r0_ref[0], wb_ref[0:HALF, :],
                    preferred_element_type=jnp.float32)
    z = z + jnp.dot(r1_ref[0], wb_ref[HALF:2 * HALF, :],
                    preferred_element_type=jnp.float32)
    z = jnp.maximum(z, z * jnp.float32(0.01))
    o_ref[...] = jnp.where(recv_ref[...] > 0.0, z, jnp.float32(0.0))


def _recv_red_body(recv_ref, o_ref):
    o_ref[...] = jnp.max(recv_ref[...], axis=0)[:, None]


def _sc_body(src_hbm, dst_hbm, nt_hbm, a2_hbm, b2_hbm, ct_hbm, cb_hbm, zrow_hbm,
             red2_hbm, recvt_hbm,
             nt_v, srcch_v, dstch_v, pk_l, eid_l,
             aidx_b, dv_b, eid_b, bidx, sidx2, recv_v,
             abuf, bbuf, cbuf, red_s, sem_a, sem_b, sem_c):
    c = lax.axis_index("c")
    s = lax.axis_index("s")
    lane = lax.iota(jnp.int32, 16)
    cNP = c * NPAD
    cE = c * E
    ones16 = jnp.full((16,), 1.0, jnp.float32)
    zero16f = jnp.zeros((16,), jnp.float32)

    pltpu.sync_copy(nt_hbm, nt_v)

    def z_body(i, _):
        recv_v[pl.ds(i * 16, 16)] = zero16f
        return 0

    lax.fori_loop(0, NRP // 16, z_body, 0)

    stripe_base = s * STRIPE

    def chunk_body(j, kk):
        ebase = stripe_base + j * CCH
        pltpu.sync_copy(src_hbm.at[pl.ds(ebase, CCH)], srcch_v)
        pltpu.sync_copy(dst_hbm.at[pl.ds(ebase, CCH)], dstch_v)

        def vec_body(i, kk):
            srcv = srcch_v[pl.ds(i * 16, 16)]
            dstv = dstch_v[pl.ds(i * 16, 16)]
            w = plsc.load_gather(nt_v, [jnp.right_shift(srcv, 4)])
            sh = jnp.left_shift(jnp.bitwise_and(srcv, 15), 1)
            nt = jnp.bitwise_and(lax.shift_right_logical(w, sh), 3)
            m = nt == TASK
            plsc.store_scatter(recv_v, [dstv], ones16, mask=m)
            eidv = (ebase + i * 16) + lane
            pkv = jnp.bitwise_or(jnp.left_shift(srcv, SHIFT), dstv)
            plsc.store_compressed(pk_l.at[pl.ds(kk, 16)], pkv, mask=m)
            plsc.store_compressed(eid_l.at[pl.ds(kk, 16)], eidv, mask=m)
            return kk + jnp.sum(m.astype(jnp.int32))

        return lax.fori_loop(0, CCH // 16, vec_body, kk)

    kk = lax.fori_loop(0, STRIPE // CCH, chunk_body, jnp.int32(0))

    pk_l[pl.ds(kk, 16)] = jnp.full((16,), SENT, jnp.int32)
    eid_l[pl.ds(kk, 16)] = jnp.zeros((16,), jnp.int32)

    @pl.when(c == 0)
    def _():
        pltpu.sync_copy(recv_v, recvt_hbm.at[pl.ds(s * NRP, NRP)])

    nv = (kk + 15) // 16

    for p in range(2):
        @pl.when(s < NH // ZR)
        def _():
            pltpu.sync_copy(zrow_hbm, red_s.at[pl.ds(s * ZR, ZR)])

        plsc.subcore_barrier()
        dbase = p * NH + cNP

        def batch():
            for t in range(GCH // 16):
                sl = pl.ds(t * 16, 16)
                dv = dv_b[sl]
                sidx2[0, sl] = dv
                bidx[sl] = dv + dbase
            ca = pltpu.async_copy(a2_hbm.at[aidx_b.at[pl.ds(0, GCH)]],
                                  abuf, sem_a)
            cb = pltpu.async_copy(b2_hbm.at[bidx], bbuf, sem_b)
            @pl.when(c == 0)
            def _():
                pltpu.async_copy(ct_hbm.at[eid_b.at[pl.ds(0, GCH)]],
                                 cbuf, sem_c)

            @pl.when(c == 1)
            def _():
                pltpu.async_copy(cb_hbm.at[eid_b.at[pl.ds(0, GCH)]],
                                 cbuf, sem_c)

            ca.wait()
            cb.wait()
            pltpu.make_async_copy(ct_hbm.at[eid_b.at[pl.ds(0, GCH)]],
                                  cbuf, sem_c).wait()

            def row_body(i, _):
                for k2 in range(HALF // 16):
                    sl = pl.ds(k2 * 16, 16)
                    v = abuf[i, sl] + bbuf[i, sl] + cbuf[i, sl]
                    abuf[i, sl] = jnp.maximum(v, v * jnp.float32(0.01))
                return 0

            lax.fori_loop(0, GCH, row_body, 0)
            pltpu.sync_copy(abuf, red_s.at[sidx2.at[0]], add=True)

        def cnt_body(i, kp):
            pkv = pk_l[pl.ds(i * 16, 16)]
            dv = jnp.bitwise_and(pkv, SENT)
            dvl = dv - p * NH
            mr = (dvl >= 0) & (dvl < NH)
            return kp + jnp.sum(mr.astype(jnp.int32))

        kp = lax.fori_loop(0, nv, cnt_body, jnp.int32(0))
        nbatches = (kp + GCH - 1) // GCH

        def b_body(_, st):
            i, fill = st

            def f_cond(st2):
                i2, f2 = st2
                return (f2 < GCH) & (i2 < nv)

            def f_step(st2):
                i2, f2 = st2
                pkv = pk_l[pl.ds(i2 * 16, 16)]
                ev = eid_l[pl.ds(i2 * 16, 16)]
                sv = jnp.right_shift(pkv, SHIFT)
                dv = jnp.bitwise_and(pkv, SENT)
                dvl = dv - p * NH
                mr = (dvl >= 0) & (dvl < NH)
                plsc.store_compressed(aidx_b.at[pl.ds(f2, 16)], sv + cNP,
                                      mask=mr)
                plsc.store_compressed(dv_b.at[pl.ds(f2, 16)], dvl, mask=mr)
                plsc.store_compressed(eid_b.at[pl.ds(f2, 16)], ev, mask=mr)
                return (i2 + 1, f2 + jnp.sum(mr.astype(jnp.int32)))

            i, fill = lax.while_loop(f_cond, f_step, (i, fill))

            for t in range(GCH // 16):
                aidx_b[pl.ds(fill + t * 16, 16)] = jnp.full((16,), cNP,
                                                            jnp.int32)
                dv_b[pl.ds(fill + t * 16, 16)] = jnp.full((16,), NH,
                                                          jnp.int32)
                eid_b[pl.ds(fill + t * 16, 16)] = jnp.full((16,), cE,
                                                           jnp.int32)
            batch()
            aidx_b[pl.ds(0, 16)] = aidx_b[pl.ds(GCH, 16)]
            dv_b[pl.ds(0, 16)] = dv_b[pl.ds(GCH, 16)]
            eid_b[pl.ds(0, 16)] = eid_b[pl.ds(GCH, 16)]
            return (i, jnp.maximum(fill - GCH, 0))

        lax.fori_loop(0, nbatches, b_body, (jnp.int32(0), jnp.int32(0)))

        plsc.subcore_barrier()

        @pl.when(s < NH // ZR)
        def _():
            pltpu.sync_copy(red_s.at[pl.ds(s * ZR, ZR)],
                            red2_hbm.at[pl.ds(c * NPAD + p * NH + s * ZR,
                                              ZR)])

        plsc.subcore_barrier()


def _make_sc_kernel():
    mesh = plsc.VectorSubcoreMesh(core_axis_name="c", subcore_axis_name="s",
                                  num_cores=2, num_subcores=NS)
    return pl.kernel(
        _sc_body,
        out_type=(
            jax.ShapeDtypeStruct((2 * NPAD, HALF), jnp.float32),
            jax.ShapeDtypeStruct((NS * NRP,), jnp.float32),
        ),
        mesh=mesh,
        compiler_params=pltpu.CompilerParams(needs_layout_passes=False),
        scratch_types=[
            pltpu.VMEM((NTW,), jnp.int32),
            pltpu.VMEM((CCH,), jnp.int32),
            pltpu.VMEM((CCH,), jnp.int32),
            pltpu.VMEM((CAP,), jnp.int32),
            pltpu.VMEM((CAP,), jnp.int32),
            pltpu.VMEM((SB,), jnp.int32),
            pltpu.VMEM((SB,), jnp.int32),
            pltpu.VMEM((SB,), jnp.int32),
            pltpu.VMEM((GCH,), jnp.int32),
            pltpu.VMEM((1, GCH), jnp.int32),
            pltpu.VMEM((NRP,), jnp.float32),
            pltpu.VMEM((GCH, HALF), jnp.float32),
            pltpu.VMEM((GCH, HALF), jnp.float32),
            pltpu.VMEM((GCH, HALF), jnp.float32),
            pltpu.VMEM_SHARED((NHP, HALF), jnp.float32),
            pltpu.SemaphoreType.DMA,
            pltpu.SemaphoreType.DMA,
            pltpu.SemaphoreType.DMA,
        ],
    )


_BN = 400
_BE = 3200


@jax.jit
def kernel(nf, ef, edge_index, node_type, W_edge, W_node):
    src = edge_index[0]
    dst = edge_index[1]
    w_cat = jnp.concatenate(
        [W_edge[0:DIN], W_edge[DIN:2 * DIN], W_node[0:DOUT]], axis=1)
    w_ef = W_edge[2 * DIN:2 * DIN + EFD]
    w_bot = W_node[DOUT:DOUT + DIN]

    ntb = node_type.astype(jnp.int32).reshape(N // 16, 16)
    ntp = jnp.sum(ntb << (2 * jnp.arange(16, dtype=jnp.int32)), axis=1)
    ntp = jnp.concatenate([ntp, jnp.zeros((NTW - N // 16,), jnp.int32)])

    a2, b2, d = pl.pallas_call(
        _mm_node_body,
        grid=(N // _BN,),
        in_specs=[
            pl.BlockSpec((_BN, DIN), lambda i: (i, 0)),
            pl.BlockSpec((DIN, 2 * DIN + DOUT), lambda i: (0, 0)),
        ],
        out_specs=[
            pl.BlockSpec((2, _BN, HALF), lambda i: (0, i, 0)),
            pl.BlockSpec((2, _BN, HALF), lambda i: (0, i, 0)),
            pl.BlockSpec((_BN, DOUT), lambda i: (i, 0)),
        ],
        out_shape=[
            jax.ShapeDtypeStruct((2, NPAD, HALF), jnp.float32),
            jax.ShapeDtypeStruct((2, NPAD, HALF), jnp.float32),
            jax.ShapeDtypeStruct((N, DOUT), jnp.float32),
        ],
    )(nf, w_cat)

    eft = ef.T
    ct, cb = pl.pallas_call(
        _mm_edge_body,
        grid=(E // _BE,),
        in_specs=[
            pl.BlockSpec((EFD, _BE), lambda i: (0, i)),
            pl.BlockSpec((EFD, DOUT), lambda i: (0, 0)),
        ],
        out_specs=[
            pl.BlockSpec((_BE, HALF), lambda i: (i, 0)),
            pl.BlockSpec((_BE, HALF), lambda i: (i, 0)),
        ],
        out_shape=[
            jax.ShapeDtypeStruct((E, HALF), jnp.float32),
            jax.ShapeDtypeStruct((E, HALF), jnp.float32),
        ],
    )(eft, w_ef)

    a2 = a2.reshape(2 * NPAD, HALF)
    b2 = b2.reshape(2 * NPAD, HALF)

    zrow = jnp.zeros((ZR, HALF), jnp.float32)
    red2, recvt = _make_sc_kernel()(
        src, dst, ntp, a2, b2, ct, cb, zrow)

    recv = pl.pallas_call(
        _recv_red_body,
        grid=(1,),
        in_specs=[pl.BlockSpec((NS, NRP), lambda i: (0, 0))],
        out_specs=pl.BlockSpec((NRP, 1), lambda i: (0, 0)),
        out_shape=jax.ShapeDtypeStruct((NRP, 1), jnp.float32),
    )(recvt.reshape(NS, NRP))

    out = pl.pallas_call(
        _node_out_body,
        grid=(N // _BN,),
        in_specs=[
            pl.BlockSpec((_BN, DOUT), lambda i: (i, 0)),
            pl.BlockSpec((1, _BN, HALF), lambda i: (0, i, 0)),
            pl.BlockSpec((1, _BN, HALF), lambda i: (1, i, 0)),
            pl.BlockSpec((DIN, DOUT), lambda i: (0, 0)),
            pl.BlockSpec((_BN, 1), lambda i: (i, 0)),
        ],
        out_specs=pl.BlockSpec((_BN, DOUT), lambda i: (i, 0)),
        out_shape=jax.ShapeDtypeStruct((N, DOUT), jnp.float32),
    )(d, red2.reshape(2, NPAD, HALF), red2.reshape(2, NPAD, HALF), w_bot,
      recv[:N])
    return out
